# baseline (speedup 1.0000x reference)
import functools

import jax
import jax.numpy as jnp
from jax.experimental import pallas as pl
from jax.experimental.pallas import tpu as pltpu

_ALPHA = 0.2


def _prologue1_body(nheads, nhid, x_ref, w_ref, asrc_ref, adst_ref,
                    haug_ref, u_ref, v_ref):
    h = jnp.dot(x_ref[...], w_ref[...], preferred_element_type=jnp.float32)
    fsrc = jnp.dot(h, asrc_ref[...], preferred_element_type=jnp.float32)
    fdst = jnp.dot(h, adst_ref[...], preferred_element_type=jnp.float32)
    u_ref[:, 0:nheads] = jnp.exp(-fsrc)
    u_ref[:, nheads:2 * nheads] = jnp.exp(-_ALPHA * fsrc)
    v_ref[:, 0:nheads] = jnp.exp(-fdst)
    v_ref[:, nheads:2 * nheads] = jnp.exp(-_ALPHA * fdst)
    w = nhid + 1
    ones = jnp.ones((h.shape[0], 1), jnp.float32)
    for hd in range(nheads):
        haug_ref[:, hd * w:hd * w + nhid] = h[:, hd * nhid:(hd + 1) * nhid]
        haug_ref[:, hd * w + nhid:hd * w + nhid + 1] = ones


def _attn1_body(nheads, nhid, nj, u_ref, vT_ref, haug_ref, adj_ref,
                x1_ref, acc_ref):
    j = pl.program_id(1)
    w = nhid + 1

    @pl.when(j == 0)
    def _init():
        acc_ref[...] = jnp.zeros_like(acc_ref)

    adjf = adj_ref[...].astype(jnp.float32)
    haug = haug_ref[...]
    for hd in range(nheads):
        p = jnp.minimum(
            u_ref[:, hd:hd + 1] * vT_ref[hd:hd + 1, :],
            u_ref[:, nheads + hd:nheads + hd + 1]
            * vT_ref[nheads + hd:nheads + hd + 1, :]) * adjf
        acc_ref[:, hd * w:(hd + 1) * w] += jnp.dot(
            p, haug[:, hd * w:(hd + 1) * w], preferred_element_type=jnp.float32)

    @pl.when(j == nj - 1)
    def _fin():
        acc = acc_ref[...]
        for hd in range(nheads):
            hp = acc[:, hd * w:hd * w + nhid]
            rs = acc[:, hd * w + nhid:hd * w + nhid + 1]
            x = hp / rs
            x1_ref[:, hd * nhid:(hd + 1) * nhid] = jnp.where(
                x > 0.0, x, jnp.exp(x) - 1.0)


def _prologue2_body(nclass, x_ref, w_ref, a2_ref, haug_ref, fe_ref):
    h2 = jnp.dot(x_ref[...], w_ref[...], preferred_element_type=jnp.float32)
    f2 = jnp.dot(h2, a2_ref[...], preferred_element_type=jnp.float32)
    fe_ref[:, 0:8] = jnp.exp(-f2)
    fe_ref[:, 8:16] = jnp.exp(-_ALPHA * f2)
    haug_ref[:, 0:nclass] = h2
    haug_ref[:, nclass:nclass + 1] = jnp.ones((h2.shape[0], 1), jnp.float32)


def _attn2_body(nclass, nj, fe_ref, feT_ref, haug_ref, adj_ref, out_ref,
                acc_ref):
    j = pl.program_id(1)

    @pl.when(j == 0)
    def _init():
        acc_ref[...] = jnp.zeros_like(acc_ref)

    adjf = adj_ref[...].astype(jnp.float32)
    p = jnp.minimum(fe_ref[:, 0:1] * feT_ref[1:2, :],
                    fe_ref[:, 8:9] * feT_ref[9:10, :]) * adjf
    acc_ref[...] += jnp.dot(p, haug_ref[...], preferred_element_type=jnp.float32)

    @pl.when(j == nj - 1)
    def _fin():
        acc = acc_ref[...]
        x = acc[:, 0:nclass] / acc[:, nclass:nclass + 1]
        x = jnp.where(x > 0.0, x, jnp.exp(x) - 1.0)
        m = jnp.max(x, axis=1, keepdims=True)
        lse = m + jnp.log(jnp.sum(jnp.exp(x - m), axis=1, keepdims=True))
        out_ref[...] = x - lse


def kernel(features, adj, Ws, As, W_out, a_out):
    n, nfeat = features.shape
    nheads, _, nhid = Ws.shape
    nclass = W_out.shape[1]
    ndim = nheads * nhid
    w1 = nhid + 1

    BI, BJ = 256, 512
    ni, nj = n // BI, n // BJ

    adj8 = (adj != 0).astype(jnp.int8)
    W_all = jnp.transpose(Ws, (1, 0, 2)).reshape(nfeat, ndim)
    eye = jnp.eye(nheads, dtype=jnp.float32)
    a_src_mat = (eye[:, None, :] * As[:, 0, :nhid][:, :, None]).reshape(ndim, nheads)
    a_dst_mat = (eye[:, None, :] * As[:, 0, nhid:][:, :, None]).reshape(ndim, nheads)
    a2_mat = jnp.pad(
        jnp.stack([a_out[0, :nclass], a_out[0, nclass:]], axis=1),
        ((0, 0), (0, 6)))

    haug, u, v = pl.pallas_call(
        functools.partial(_prologue1_body, nheads, nhid),
        grid=(ni,),
        in_specs=[
            pl.BlockSpec((BI, nfeat), lambda i: (i, 0)),
            pl.BlockSpec((nfeat, ndim), lambda i: (0, 0)),
            pl.BlockSpec((ndim, nheads), lambda i: (0, 0)),
            pl.BlockSpec((ndim, nheads), lambda i: (0, 0)),
        ],
        out_specs=[
            pl.BlockSpec((BI, nheads * w1), lambda i: (i, 0)),
            pl.BlockSpec((BI, 2 * nheads), lambda i: (i, 0)),
            pl.BlockSpec((BI, 2 * nheads), lambda i: (i, 0)),
        ],
        out_shape=[
            jax.ShapeDtypeStruct((n, nheads * w1), jnp.float32),
            jax.ShapeDtypeStruct((n, 2 * nheads), jnp.float32),
            jax.ShapeDtypeStruct((n, 2 * nheads), jnp.float32),
        ],
    )(features, W_all, a_src_mat, a_dst_mat)

    vT = v.T

    x1 = pl.pallas_call(
        functools.partial(_attn1_body, nheads, nhid, nj),
        grid=(ni, nj),
        in_specs=[
            pl.BlockSpec((BI, 2 * nheads), lambda i, j: (i, 0)),
            pl.BlockSpec((2 * nheads, BJ), lambda i, j: (0, j)),
            pl.BlockSpec((BJ, nheads * w1), lambda i, j: (j, 0)),
            pl.BlockSpec((BI, BJ), lambda i, j: (i, j)),
        ],
        out_specs=pl.BlockSpec((BI, ndim), lambda i, j: (i, 0)),
        out_shape=jax.ShapeDtypeStruct((n, ndim), jnp.float32),
        scratch_shapes=[pltpu.VMEM((BI, nheads * w1), jnp.float32)],
        compiler_params=pltpu.CompilerParams(
            dimension_semantics=("parallel", "arbitrary")),
    )(u, vT, haug, adj8)

    h2aug, fe = pl.pallas_call(
        functools.partial(_prologue2_body, nclass),
        grid=(ni,),
        in_specs=[
            pl.BlockSpec((BI, ndim), lambda i: (i, 0)),
            pl.BlockSpec((ndim, nclass), lambda i: (0, 0)),
            pl.BlockSpec((nclass, 8), lambda i: (0, 0)),
        ],
        out_specs=[
            pl.BlockSpec((BI, nclass + 1), lambda i: (i, 0)),
            pl.BlockSpec((BI, 16), lambda i: (i, 0)),
        ],
        out_shape=[
            jax.ShapeDtypeStruct((n, nclass + 1), jnp.float32),
            jax.ShapeDtypeStruct((n, 16), jnp.float32),
        ],
    )(x1, W_out, a2_mat)

    feT = fe.T

    out = pl.pallas_call(
        functools.partial(_attn2_body, nclass, nj),
        grid=(ni, nj),
        in_specs=[
            pl.BlockSpec((BI, 16), lambda i, j: (i, 0)),
            pl.BlockSpec((16, BJ), lambda i, j: (0, j)),
            pl.BlockSpec((BJ, nclass + 1), lambda i, j: (j, 0)),
            pl.BlockSpec((BI, BJ), lambda i, j: (i, j)),
        ],
        out_specs=pl.BlockSpec((BI, nclass), lambda i, j: (i, 0)),
        out_shape=jax.ShapeDtypeStruct((n, nclass), jnp.float32),
        scratch_shapes=[pltpu.VMEM((BI, nclass + 1), jnp.float32)],
        compiler_params=pltpu.CompilerParams(
            dimension_semantics=("parallel", "arbitrary")),
    )(fe, feT, h2aug, adj8)

    return out


# speedup vs baseline: 1.0480x; 1.0480x over previous
import functools

import jax
import jax.numpy as jnp
from jax.experimental import pallas as pl
from jax.experimental.pallas import tpu as pltpu

_ALPHA = 0.2


def _prologue1_body(nheads, nhid, x_ref, w_ref, asrc_ref, adst_ref,
                    haug_ref, u_ref, v_ref):
    h = jnp.dot(x_ref[...], w_ref[...], preferred_element_type=jnp.float32)
    fsrc = jnp.dot(h, asrc_ref[...], preferred_element_type=jnp.float32)
    fdst = jnp.dot(h, adst_ref[...], preferred_element_type=jnp.float32)
    u_ref[:, 0:nheads] = jnp.exp(-fsrc)
    u_ref[:, nheads:2 * nheads] = jnp.exp(-_ALPHA * fsrc)
    v_ref[:, 0:nheads] = jnp.exp(-fdst)
    v_ref[:, nheads:2 * nheads] = jnp.exp(-_ALPHA * fdst)
    w = nhid + 1
    ones = jnp.ones((h.shape[0], 1), jnp.float32)
    for hd in range(nheads):
        haug_ref[:, hd * w:hd * w + nhid] = h[:, hd * nhid:(hd + 1) * nhid]
        haug_ref[:, hd * w + nhid:hd * w + nhid + 1] = ones


def _attn1_body(nheads, nhid, nj, u_ref, vT_ref, haug_ref, adj_ref,
                x1_ref, acc_ref):
    j = pl.program_id(1)
    w = nhid + 1

    @pl.when(j == 0)
    def _init():
        acc_ref[...] = jnp.zeros_like(acc_ref)

    adjf = adj_ref[...].astype(jnp.float32)
    haug = haug_ref[...]
    for hd in range(nheads):
        p = jnp.minimum(
            u_ref[:, hd:hd + 1] * vT_ref[hd:hd + 1, :],
            u_ref[:, nheads + hd:nheads + hd + 1]
            * vT_ref[nheads + hd:nheads + hd + 1, :]) * adjf
        acc_ref[:, hd * w:(hd + 1) * w] += jnp.dot(
            p, haug[:, hd * w:(hd + 1) * w], preferred_element_type=jnp.float32)

    @pl.when(j == nj - 1)
    def _fin():
        acc = acc_ref[...]
        for hd in range(nheads):
            hp = acc[:, hd * w:hd * w + nhid]
            rs = acc[:, hd * w + nhid:hd * w + nhid + 1]
            x = hp / rs
            x1_ref[:, hd * nhid:(hd + 1) * nhid] = jnp.where(
                x > 0.0, x, jnp.exp(x) - 1.0)


def _prologue2_body(nclass, x_ref, w_ref, a2_ref, haug_ref, fe_ref):
    h2 = jnp.dot(x_ref[...], w_ref[...], preferred_element_type=jnp.float32)
    f2 = jnp.dot(h2, a2_ref[...], preferred_element_type=jnp.float32)
    fe_ref[:, 0:8] = jnp.exp(-f2)
    fe_ref[:, 8:16] = jnp.exp(-_ALPHA * f2)
    haug_ref[:, 0:nclass] = h2
    haug_ref[:, nclass:nclass + 1] = jnp.ones((h2.shape[0], 1), jnp.float32)


def _attn2_body(nclass, nj, fe_ref, feT_ref, haug_ref, adj_ref, out_ref,
                acc_ref):
    j = pl.program_id(1)

    @pl.when(j == 0)
    def _init():
        acc_ref[...] = jnp.zeros_like(acc_ref)

    adjf = adj_ref[...].astype(jnp.float32)
    p = jnp.minimum(fe_ref[:, 0:1] * feT_ref[1:2, :],
                    fe_ref[:, 8:9] * feT_ref[9:10, :]) * adjf
    acc_ref[...] += jnp.dot(p, haug_ref[...], preferred_element_type=jnp.float32)

    @pl.when(j == nj - 1)
    def _fin():
        acc = acc_ref[...]
        x = acc[:, 0:nclass] / acc[:, nclass:nclass + 1]
        x = jnp.where(x > 0.0, x, jnp.exp(x) - 1.0)
        m = jnp.max(x, axis=1, keepdims=True)
        lse = m + jnp.log(jnp.sum(jnp.exp(x - m), axis=1, keepdims=True))
        out_ref[...] = x - lse


def kernel(features, adj, Ws, As, W_out, a_out):
    n, nfeat = features.shape
    nheads, _, nhid = Ws.shape
    nclass = W_out.shape[1]
    ndim = nheads * nhid
    w1 = nhid + 1

    BI, BJ = 256, 512
    ni, nj = n // BI, n // BJ

    adj8 = adj
    W_all = jnp.transpose(Ws, (1, 0, 2)).reshape(nfeat, ndim)
    eye = jnp.eye(nheads, dtype=jnp.float32)
    a_src_mat = (eye[:, None, :] * As[:, 0, :nhid][:, :, None]).reshape(ndim, nheads)
    a_dst_mat = (eye[:, None, :] * As[:, 0, nhid:][:, :, None]).reshape(ndim, nheads)
    a2_mat = jnp.pad(
        jnp.stack([a_out[0, :nclass], a_out[0, nclass:]], axis=1),
        ((0, 0), (0, 6)))

    haug, u, v = pl.pallas_call(
        functools.partial(_prologue1_body, nheads, nhid),
        grid=(ni,),
        in_specs=[
            pl.BlockSpec((BI, nfeat), lambda i: (i, 0)),
            pl.BlockSpec((nfeat, ndim), lambda i: (0, 0)),
            pl.BlockSpec((ndim, nheads), lambda i: (0, 0)),
            pl.BlockSpec((ndim, nheads), lambda i: (0, 0)),
        ],
        out_specs=[
            pl.BlockSpec((BI, nheads * w1), lambda i: (i, 0)),
            pl.BlockSpec((BI, 2 * nheads), lambda i: (i, 0)),
            pl.BlockSpec((BI, 2 * nheads), lambda i: (i, 0)),
        ],
        out_shape=[
            jax.ShapeDtypeStruct((n, nheads * w1), jnp.float32),
            jax.ShapeDtypeStruct((n, 2 * nheads), jnp.float32),
            jax.ShapeDtypeStruct((n, 2 * nheads), jnp.float32),
        ],
    )(features, W_all, a_src_mat, a_dst_mat)

    vT = v.T

    x1 = pl.pallas_call(
        functools.partial(_attn1_body, nheads, nhid, nj),
        grid=(ni, nj),
        in_specs=[
            pl.BlockSpec((BI, 2 * nheads), lambda i, j: (i, 0)),
            pl.BlockSpec((2 * nheads, BJ), lambda i, j: (0, j)),
            pl.BlockSpec((BJ, nheads * w1), lambda i, j: (j, 0)),
            pl.BlockSpec((BI, BJ), lambda i, j: (i, j)),
        ],
        out_specs=pl.BlockSpec((BI, ndim), lambda i, j: (i, 0)),
        out_shape=jax.ShapeDtypeStruct((n, ndim), jnp.float32),
        scratch_shapes=[pltpu.VMEM((BI, nheads * w1), jnp.float32)],
        compiler_params=pltpu.CompilerParams(
            dimension_semantics=("parallel", "arbitrary")),
    )(u, vT, haug, adj8)

    h2aug, fe = pl.pallas_call(
        functools.partial(_prologue2_body, nclass),
        grid=(ni,),
        in_specs=[
            pl.BlockSpec((BI, ndim), lambda i: (i, 0)),
            pl.BlockSpec((ndim, nclass), lambda i: (0, 0)),
            pl.BlockSpec((nclass, 8), lambda i: (0, 0)),
        ],
        out_specs=[
            pl.BlockSpec((BI, nclass + 1), lambda i: (i, 0)),
            pl.BlockSpec((BI, 16), lambda i: (i, 0)),
        ],
        out_shape=[
            jax.ShapeDtypeStruct((n, nclass + 1), jnp.float32),
            jax.ShapeDtypeStruct((n, 16), jnp.float32),
        ],
    )(x1, W_out, a2_mat)

    feT = fe.T

    out = pl.pallas_call(
        functools.partial(_attn2_body, nclass, nj),
        grid=(ni, nj),
        in_specs=[
            pl.BlockSpec((BI, 16), lambda i, j: (i, 0)),
            pl.BlockSpec((16, BJ), lambda i, j: (0, j)),
            pl.BlockSpec((BJ, nclass + 1), lambda i, j: (j, 0)),
            pl.BlockSpec((BI, BJ), lambda i, j: (i, j)),
        ],
        out_specs=pl.BlockSpec((BI, nclass), lambda i, j: (i, 0)),
        out_shape=jax.ShapeDtypeStruct((n, nclass), jnp.float32),
        scratch_shapes=[pltpu.VMEM((BI, nclass + 1), jnp.float32)],
        compiler_params=pltpu.CompilerParams(
            dimension_semantics=("parallel", "arbitrary")),
    )(fe, feT, h2aug, adj8)

    return out


# speedup vs baseline: 1.7106x; 1.6323x over previous
import functools

import jax
import jax.numpy as jnp
from jax.experimental import pallas as pl
from jax.experimental.pallas import tpu as pltpu

_ALPHA = 0.2


def _prologue1_body(nheads, nhid, x_ref, w_ref, asrc_ref, adst_ref,
                    haug_ref, fsrc_ref, fdst_ref):
    h = jnp.dot(x_ref[...], w_ref[...], preferred_element_type=jnp.float32)
    fsrc_ref[...] = jnp.dot(h, asrc_ref[...], preferred_element_type=jnp.float32)
    fdst_ref[...] = jnp.dot(h, adst_ref[...], preferred_element_type=jnp.float32)
    w = nhid + 1
    ones = jnp.ones((h.shape[0], 1), jnp.float32)
    for hd in range(nheads):
        haug_ref[:, hd * w:hd * w + nhid] = h[:, hd * nhid:(hd + 1) * nhid]
        haug_ref[:, hd * w + nhid:hd * w + nhid + 1] = ones


def _attn1_body(nheads, nhid, nj, fsrc_ref, fdstT_ref, haug_ref, adj_ref,
                x1_ref, acc_ref):
    j = pl.program_id(1)
    w = nhid + 1

    @pl.when(j == 0)
    def _init():
        acc_ref[...] = jnp.zeros_like(acc_ref)

    adjf = adj_ref[...]
    haug = haug_ref[...]
    for hd in range(nheads):
        s = fsrc_ref[:, hd:hd + 1] + fdstT_ref[hd:hd + 1, :]
        m = jnp.minimum(s, _ALPHA * s)
        p = jnp.exp(m) * adjf
        acc_ref[:, hd * w:(hd + 1) * w] += jnp.dot(
            p, haug[:, hd * w:(hd + 1) * w], preferred_element_type=jnp.float32)

    @pl.when(j == nj - 1)
    def _fin():
        acc = acc_ref[...]
        for hd in range(nheads):
            hp = acc[:, hd * w:hd * w + nhid]
            rs = acc[:, hd * w + nhid:hd * w + nhid + 1]
            x = hp / rs
            x1_ref[:, hd * nhid:(hd + 1) * nhid] = jnp.where(
                x > 0.0, x, jnp.exp(x) - 1.0)


def _prologue2_body(nclass, x_ref, w_ref, a2_ref, haug_ref, f2_ref):
    h2 = jnp.dot(x_ref[...], w_ref[...], preferred_element_type=jnp.float32)
    f2_ref[...] = jnp.dot(h2, a2_ref[...], preferred_element_type=jnp.float32)
    haug_ref[:, 0:nclass] = h2
    haug_ref[:, nclass:nclass + 1] = jnp.ones((h2.shape[0], 1), jnp.float32)


def _attn2_body(nclass, nj, f2_ref, f2T_ref, haug_ref, adj_ref, out_ref,
                acc_ref):
    j = pl.program_id(1)

    @pl.when(j == 0)
    def _init():
        acc_ref[...] = jnp.zeros_like(acc_ref)

    adjf = adj_ref[...]
    s = f2_ref[:, 0:1] + f2T_ref[1:2, :]
    m = jnp.minimum(s, _ALPHA * s)
    p = jnp.exp(m) * adjf
    acc_ref[...] += jnp.dot(p, haug_ref[...], preferred_element_type=jnp.float32)

    @pl.when(j == nj - 1)
    def _fin():
        acc = acc_ref[...]
        x = acc[:, 0:nclass] / acc[:, nclass:nclass + 1]
        x = jnp.where(x > 0.0, x, jnp.exp(x) - 1.0)
        m = jnp.max(x, axis=1, keepdims=True)
        lse = m + jnp.log(jnp.sum(jnp.exp(x - m), axis=1, keepdims=True))
        out_ref[...] = x - lse


def kernel(features, adj, Ws, As, W_out, a_out):
    n, nfeat = features.shape
    nheads, _, nhid = Ws.shape
    nclass = W_out.shape[1]
    ndim = nheads * nhid
    w1 = nhid + 1

    BI, BJ = 512, 1024
    ni, nj = n // BI, n // BJ

    W_all = jnp.transpose(Ws, (1, 0, 2)).reshape(nfeat, ndim)
    eye = jnp.eye(nheads, dtype=jnp.float32)
    a_src_mat = -(eye[:, None, :] * As[:, 0, :nhid][:, :, None]).reshape(ndim, nheads)
    a_dst_mat = -(eye[:, None, :] * As[:, 0, nhid:][:, :, None]).reshape(ndim, nheads)
    a2_mat = -jnp.pad(
        jnp.stack([a_out[0, :nclass], a_out[0, nclass:]], axis=1),
        ((0, 0), (0, 6)))
    adjf = adj.astype(jnp.float32)

    haug, fsrc, fdst = pl.pallas_call(
        functools.partial(_prologue1_body, nheads, nhid),
        grid=(ni,),
        in_specs=[
            pl.BlockSpec((BI, nfeat), lambda i: (i, 0)),
            pl.BlockSpec((nfeat, ndim), lambda i: (0, 0)),
            pl.BlockSpec((ndim, nheads), lambda i: (0, 0)),
            pl.BlockSpec((ndim, nheads), lambda i: (0, 0)),
        ],
        out_specs=[
            pl.BlockSpec((BI, nheads * w1), lambda i: (i, 0)),
            pl.BlockSpec((BI, nheads), lambda i: (i, 0)),
            pl.BlockSpec((BI, nheads), lambda i: (i, 0)),
        ],
        out_shape=[
            jax.ShapeDtypeStruct((n, nheads * w1), jnp.float32),
            jax.ShapeDtypeStruct((n, nheads), jnp.float32),
            jax.ShapeDtypeStruct((n, nheads), jnp.float32),
        ],
    )(features, W_all, a_src_mat, a_dst_mat)

    fdstT = fdst.T

    x1 = pl.pallas_call(
        functools.partial(_attn1_body, nheads, nhid, nj),
        grid=(ni, nj),
        in_specs=[
            pl.BlockSpec((BI, nheads), lambda i, j: (i, 0)),
            pl.BlockSpec((nheads, BJ), lambda i, j: (0, j)),
            pl.BlockSpec((BJ, nheads * w1), lambda i, j: (j, 0)),
            pl.BlockSpec((BI, BJ), lambda i, j: (i, j)),
        ],
        out_specs=pl.BlockSpec((BI, ndim), lambda i, j: (i, 0)),
        out_shape=jax.ShapeDtypeStruct((n, ndim), jnp.float32),
        scratch_shapes=[pltpu.VMEM((BI, nheads * w1), jnp.float32)],
        compiler_params=pltpu.CompilerParams(
            dimension_semantics=("parallel", "arbitrary")),
    )(fsrc, fdstT, haug, adjf)

    h2aug, f2 = pl.pallas_call(
        functools.partial(_prologue2_body, nclass),
        grid=(ni,),
        in_specs=[
            pl.BlockSpec((BI, ndim), lambda i: (i, 0)),
            pl.BlockSpec((ndim, nclass), lambda i: (0, 0)),
            pl.BlockSpec((nclass, 8), lambda i: (0, 0)),
        ],
        out_specs=[
            pl.BlockSpec((BI, nclass + 1), lambda i: (i, 0)),
            pl.BlockSpec((BI, 8), lambda i: (i, 0)),
        ],
        out_shape=[
            jax.ShapeDtypeStruct((n, nclass + 1), jnp.float32),
            jax.ShapeDtypeStruct((n, 8), jnp.float32),
        ],
    )(x1, W_out, a2_mat)

    f2T = f2.T

    out = pl.pallas_call(
        functools.partial(_attn2_body, nclass, nj),
        grid=(ni, nj),
        in_specs=[
            pl.BlockSpec((BI, 8), lambda i, j: (i, 0)),
            pl.BlockSpec((8, BJ), lambda i, j: (0, j)),
            pl.BlockSpec((BJ, nclass + 1), lambda i, j: (j, 0)),
            pl.BlockSpec((BI, BJ), lambda i, j: (i, j)),
        ],
        out_specs=pl.BlockSpec((BI, nclass), lambda i, j: (i, 0)),
        out_shape=jax.ShapeDtypeStruct((n, nclass), jnp.float32),
        scratch_shapes=[pltpu.VMEM((BI, nclass + 1), jnp.float32)],
        compiler_params=pltpu.CompilerParams(
            dimension_semantics=("parallel", "arbitrary")),
    )(f2, f2T, h2aug, adjf)

    return out


# speedup vs baseline: 2.1501x; 1.2569x over previous
import functools

import jax
import jax.numpy as jnp
from jax.experimental import pallas as pl
from jax.experimental.pallas import tpu as pltpu

_ALPHA = 0.2


def _prologue1_body(nheads, nhid, x_ref, w_ref, asrc_ref, adst_ref,
                    haug_ref, fsrc_ref, fdstT_ref):
    h = jnp.dot(x_ref[...], w_ref[...], preferred_element_type=jnp.float32)
    fsrc_ref[...] = jnp.dot(h, asrc_ref[...], preferred_element_type=jnp.float32)
    fdstT_ref[...] = jnp.transpose(
        jnp.dot(h, adst_ref[...], preferred_element_type=jnp.float32))
    w = nhid + 1
    ones = jnp.ones((h.shape[0], 1), jnp.float32)
    for hd in range(nheads):
        haug_ref[:, hd * w:hd * w + nhid] = h[:, hd * nhid:(hd + 1) * nhid]
        haug_ref[:, hd * w + nhid:hd * w + nhid + 1] = ones


def _attn1_body(nheads, nhid, nclass, nj, fsrc_ref, fdstT_ref, haug_ref,
                adj_ref, w2_ref, a2_ref, h2aug_ref, f2_ref, f2T_ref, acc_ref):
    j = pl.program_id(1)
    w = nhid + 1

    @pl.when(j == 0)
    def _init():
        acc_ref[...] = jnp.zeros_like(acc_ref)

    adjf = adj_ref[...].astype(jnp.float32)
    haug = haug_ref[...]
    for hd in range(nheads):
        s = fsrc_ref[:, hd:hd + 1] + fdstT_ref[hd:hd + 1, :]
        m = jnp.minimum(s, _ALPHA * s)
        p = jnp.exp(m) * adjf
        acc_ref[:, hd * w:(hd + 1) * w] += jnp.dot(
            p, haug[:, hd * w:(hd + 1) * w], preferred_element_type=jnp.float32)

    @pl.when(j == nj - 1)
    def _fin():
        acc = acc_ref[...]
        cols = []
        for hd in range(nheads):
            hp = acc[:, hd * w:hd * w + nhid]
            rs = acc[:, hd * w + nhid:hd * w + nhid + 1]
            x = hp / rs
            cols.append(jnp.where(x > 0.0, x, jnp.exp(x) - 1.0))
        x1 = jnp.concatenate(cols, axis=1)
        h2 = jnp.dot(x1, w2_ref[...], preferred_element_type=jnp.float32)
        f2 = jnp.dot(h2, a2_ref[...], preferred_element_type=jnp.float32)
        f2_ref[...] = f2
        f2T_ref[...] = jnp.transpose(f2)
        h2aug_ref[:, 0:nclass] = h2
        h2aug_ref[:, nclass:nclass + 1] = jnp.ones((h2.shape[0], 1),
                                                   jnp.float32)


def _attn2_body(nclass, nj, f2_ref, f2T_ref, haug_ref, adj_ref, out_ref,
                acc_ref):
    j = pl.program_id(1)

    @pl.when(j == 0)
    def _init():
        acc_ref[...] = jnp.zeros_like(acc_ref)

    adjf = adj_ref[...].astype(jnp.float32)
    s = f2_ref[:, 0:1] + f2T_ref[1:2, :]
    m = jnp.minimum(s, _ALPHA * s)
    p = jnp.exp(m) * adjf
    acc_ref[...] += jnp.dot(p, haug_ref[...], preferred_element_type=jnp.float32)

    @pl.when(j == nj - 1)
    def _fin():
        acc = acc_ref[...]
        x = acc[:, 0:nclass] / acc[:, nclass:nclass + 1]
        x = jnp.where(x > 0.0, x, jnp.exp(x) - 1.0)
        m = jnp.max(x, axis=1, keepdims=True)
        lse = m + jnp.log(jnp.sum(jnp.exp(x - m), axis=1, keepdims=True))
        out_ref[...] = x - lse


def kernel(features, adj, Ws, As, W_out, a_out):
    n, nfeat = features.shape
    nheads, _, nhid = Ws.shape
    nclass = W_out.shape[1]
    ndim = nheads * nhid
    w1 = nhid + 1

    BI, BJ = 512, 1024
    ni, nj = n // BI, n // BJ

    W_all = jnp.transpose(Ws, (1, 0, 2)).reshape(nfeat, ndim)
    eye = jnp.eye(nheads, dtype=jnp.float32)
    a_src_mat = -(eye[:, None, :] * As[:, 0, :nhid][:, :, None]).reshape(ndim, nheads)
    a_dst_mat = -(eye[:, None, :] * As[:, 0, nhid:][:, :, None]).reshape(ndim, nheads)
    a2_mat = -jnp.pad(
        jnp.stack([a_out[0, :nclass], a_out[0, nclass:]], axis=1),
        ((0, 0), (0, 6)))

    haug, fsrc, fdstT = pl.pallas_call(
        functools.partial(_prologue1_body, nheads, nhid),
        grid=(ni,),
        in_specs=[
            pl.BlockSpec((BI, nfeat), lambda i: (i, 0)),
            pl.BlockSpec((nfeat, ndim), lambda i: (0, 0)),
            pl.BlockSpec((ndim, nheads), lambda i: (0, 0)),
            pl.BlockSpec((ndim, nheads), lambda i: (0, 0)),
        ],
        out_specs=[
            pl.BlockSpec((BI, nheads * w1), lambda i: (i, 0)),
            pl.BlockSpec((BI, nheads), lambda i: (i, 0)),
            pl.BlockSpec((nheads, BI), lambda i: (0, i)),
        ],
        out_shape=[
            jax.ShapeDtypeStruct((n, nheads * w1), jnp.float32),
            jax.ShapeDtypeStruct((n, nheads), jnp.float32),
            jax.ShapeDtypeStruct((nheads, n), jnp.float32),
        ],
    )(features, W_all, a_src_mat, a_dst_mat)

    h2aug, f2, f2T = pl.pallas_call(
        functools.partial(_attn1_body, nheads, nhid, nclass, nj),
        grid=(ni, nj),
        in_specs=[
            pl.BlockSpec((BI, nheads), lambda i, j: (i, 0)),
            pl.BlockSpec((nheads, BJ), lambda i, j: (0, j)),
            pl.BlockSpec((BJ, nheads * w1), lambda i, j: (j, 0)),
            pl.BlockSpec((BI, BJ), lambda i, j: (i, j)),
            pl.BlockSpec((ndim, nclass), lambda i, j: (0, 0)),
            pl.BlockSpec((nclass, 8), lambda i, j: (0, 0)),
        ],
        out_specs=[
            pl.BlockSpec((BI, nclass + 1), lambda i, j: (i, 0)),
            pl.BlockSpec((BI, 8), lambda i, j: (i, 0)),
            pl.BlockSpec((8, BI), lambda i, j: (0, i)),
        ],
        out_shape=[
            jax.ShapeDtypeStruct((n, nclass + 1), jnp.float32),
            jax.ShapeDtypeStruct((n, 8), jnp.float32),
            jax.ShapeDtypeStruct((8, n), jnp.float32),
        ],
        scratch_shapes=[pltpu.VMEM((BI, nheads * w1), jnp.float32)],
        compiler_params=pltpu.CompilerParams(
            dimension_semantics=("parallel", "arbitrary")),
    )(fsrc, fdstT, haug, adj, W_out, a2_mat)

    out = pl.pallas_call(
        functools.partial(_attn2_body, nclass, nj),
        grid=(ni, nj),
        in_specs=[
            pl.BlockSpec((BI, 8), lambda i, j: (i, 0)),
            pl.BlockSpec((8, BJ), lambda i, j: (0, j)),
            pl.BlockSpec((BJ, nclass + 1), lambda i, j: (j, 0)),
            pl.BlockSpec((BI, BJ), lambda i, j: (i, j)),
        ],
        out_specs=pl.BlockSpec((BI, nclass), lambda i, j: (i, 0)),
        out_shape=jax.ShapeDtypeStruct((n, nclass), jnp.float32),
        scratch_shapes=[pltpu.VMEM((BI, nclass + 1), jnp.float32)],
        compiler_params=pltpu.CompilerParams(
            dimension_semantics=("parallel", "arbitrary")),
    )(f2, f2T, h2aug, adj)

    return out


# speedup vs baseline: 2.6901x; 1.2511x over previous
import functools

import jax
import jax.numpy as jnp
from jax.experimental import pallas as pl
from jax.experimental.pallas import tpu as pltpu

_ALPHA = 0.2


def _prologue1_body(nheads, nhid, x_ref, w_ref, asrc_ref, adst_ref,
                    haug_ref, fsrc_ref, fdstT_ref):
    h = jnp.dot(x_ref[...], w_ref[...], preferred_element_type=jnp.float32)
    fsrc_ref[...] = jnp.dot(
        h, asrc_ref[...], preferred_element_type=jnp.float32).astype(jnp.bfloat16)
    fdstT_ref[...] = jnp.transpose(
        jnp.dot(h, adst_ref[...],
                preferred_element_type=jnp.float32)).astype(jnp.bfloat16)
    w = nhid + 1
    ones = jnp.ones((h.shape[0], 1), jnp.float32)
    for hd in range(nheads):
        haug_ref[:, hd * w:hd * w + nhid] = h[:, hd * nhid:(hd + 1) * nhid].astype(jnp.bfloat16)
        haug_ref[:, hd * w + nhid:hd * w + nhid + 1] = ones.astype(jnp.bfloat16)


def _attn1_body(nheads, nhid, nclass, nj, fsrc_ref, fdstT_ref, haug_ref,
                adj_ref, w2_ref, a2_ref, h2aug_ref, f2_ref, f2T_ref, acc_ref):
    j = pl.program_id(1)
    w = nhid + 1

    @pl.when(j == 0)
    def _init():
        acc_ref[...] = jnp.zeros_like(acc_ref)

    adjf = adj_ref[...].astype(jnp.bfloat16)
    haug = haug_ref[...]
    alpha = jnp.bfloat16(_ALPHA)
    for hd in range(nheads):
        s = fsrc_ref[:, hd:hd + 1] + fdstT_ref[hd:hd + 1, :]
        m = jnp.minimum(s, alpha * s)
        p = jnp.exp(m) * adjf
        acc_ref[:, hd * w:(hd + 1) * w] += jnp.dot(
            p, haug[:, hd * w:(hd + 1) * w], preferred_element_type=jnp.float32)

    @pl.when(j == nj - 1)
    def _fin():
        acc = acc_ref[...]
        cols = []
        for hd in range(nheads):
            hp = acc[:, hd * w:hd * w + nhid]
            rs = acc[:, hd * w + nhid:hd * w + nhid + 1]
            x = hp / rs
            cols.append(jnp.where(x > 0.0, x, jnp.exp(x) - 1.0))
        x1 = jnp.concatenate(cols, axis=1)
        h2 = jnp.dot(x1, w2_ref[...], preferred_element_type=jnp.float32)
        f2 = jnp.dot(h2, a2_ref[...], preferred_element_type=jnp.float32)
        f2_ref[...] = f2
        f2T_ref[...] = jnp.transpose(f2)
        h2aug_ref[:, 0:nclass] = h2
        h2aug_ref[:, nclass:nclass + 1] = jnp.ones((h2.shape[0], 1),
                                                   jnp.float32)


def _attn2_body(nclass, nj, f2_ref, f2T_ref, haug_ref, adj_ref, out_ref,
                acc_ref):
    j = pl.program_id(1)

    @pl.when(j == 0)
    def _init():
        acc_ref[...] = jnp.zeros_like(acc_ref)

    adjf = adj_ref[...].astype(jnp.float32)
    s = f2_ref[:, 0:1] + f2T_ref[1:2, :]
    m = jnp.minimum(s, _ALPHA * s)
    p = jnp.exp(m) * adjf
    acc_ref[...] += jnp.dot(p, haug_ref[...], preferred_element_type=jnp.float32)

    @pl.when(j == nj - 1)
    def _fin():
        acc = acc_ref[...]
        x = acc[:, 0:nclass] / acc[:, nclass:nclass + 1]
        x = jnp.where(x > 0.0, x, jnp.exp(x) - 1.0)
        m = jnp.max(x, axis=1, keepdims=True)
        lse = m + jnp.log(jnp.sum(jnp.exp(x - m), axis=1, keepdims=True))
        out_ref[...] = x - lse


def kernel(features, adj, Ws, As, W_out, a_out):
    n, nfeat = features.shape
    nheads, _, nhid = Ws.shape
    nclass = W_out.shape[1]
    ndim = nheads * nhid
    w1 = nhid + 1

    BI, BJ = 1024, 2048
    ni, nj = n // BI, n // BJ

    W_all = jnp.transpose(Ws, (1, 0, 2)).reshape(nfeat, ndim)
    eye = jnp.eye(nheads, dtype=jnp.float32)
    a_src_mat = -(eye[:, None, :] * As[:, 0, :nhid][:, :, None]).reshape(ndim, nheads)
    a_dst_mat = -(eye[:, None, :] * As[:, 0, nhid:][:, :, None]).reshape(ndim, nheads)
    a2_mat = -jnp.pad(
        jnp.stack([a_out[0, :nclass], a_out[0, nclass:]], axis=1),
        ((0, 0), (0, 6)))

    haug, fsrc, fdstT = pl.pallas_call(
        functools.partial(_prologue1_body, nheads, nhid),
        grid=(ni,),
        in_specs=[
            pl.BlockSpec((BI, nfeat), lambda i: (i, 0)),
            pl.BlockSpec((nfeat, ndim), lambda i: (0, 0)),
            pl.BlockSpec((ndim, nheads), lambda i: (0, 0)),
            pl.BlockSpec((ndim, nheads), lambda i: (0, 0)),
        ],
        out_specs=[
            pl.BlockSpec((BI, nheads * w1), lambda i: (i, 0)),
            pl.BlockSpec((BI, nheads), lambda i: (i, 0)),
            pl.BlockSpec((nheads, BI), lambda i: (0, i)),
        ],
        out_shape=[
            jax.ShapeDtypeStruct((n, nheads * w1), jnp.bfloat16),
            jax.ShapeDtypeStruct((n, nheads), jnp.bfloat16),
            jax.ShapeDtypeStruct((nheads, n), jnp.bfloat16),
        ],
    )(features, W_all, a_src_mat, a_dst_mat)

    h2aug, f2, f2T = pl.pallas_call(
        functools.partial(_attn1_body, nheads, nhid, nclass, nj),
        grid=(ni, nj),
        in_specs=[
            pl.BlockSpec((BI, nheads), lambda i, j: (i, 0)),
            pl.BlockSpec((nheads, BJ), lambda i, j: (0, j)),
            pl.BlockSpec((BJ, nheads * w1), lambda i, j: (j, 0)),
            pl.BlockSpec((BI, BJ), lambda i, j: (i, j)),
            pl.BlockSpec((ndim, nclass), lambda i, j: (0, 0)),
            pl.BlockSpec((nclass, 8), lambda i, j: (0, 0)),
        ],
        out_specs=[
            pl.BlockSpec((BI, nclass + 1), lambda i, j: (i, 0)),
            pl.BlockSpec((BI, 8), lambda i, j: (i, 0)),
            pl.BlockSpec((8, BI), lambda i, j: (0, i)),
        ],
        out_shape=[
            jax.ShapeDtypeStruct((n, nclass + 1), jnp.float32),
            jax.ShapeDtypeStruct((n, 8), jnp.float32),
            jax.ShapeDtypeStruct((8, n), jnp.float32),
        ],
        scratch_shapes=[pltpu.VMEM((BI, nheads * w1), jnp.float32)],
        compiler_params=pltpu.CompilerParams(
            dimension_semantics=("parallel", "arbitrary")),
    )(fsrc, fdstT, haug, adj, W_out, a2_mat)

    out = pl.pallas_call(
        functools.partial(_attn2_body, nclass, nj),
        grid=(ni, nj),
        in_specs=[
            pl.BlockSpec((BI, 8), lambda i, j: (i, 0)),
            pl.BlockSpec((8, BJ), lambda i, j: (0, j)),
            pl.BlockSpec((BJ, nclass + 1), lambda i, j: (j, 0)),
            pl.BlockSpec((BI, BJ), lambda i, j: (i, j)),
        ],
        out_specs=pl.BlockSpec((BI, nclass), lambda i, j: (i, 0)),
        out_shape=jax.ShapeDtypeStruct((n, nclass), jnp.float32),
        scratch_shapes=[pltpu.VMEM((BI, nclass + 1), jnp.float32)],
        compiler_params=pltpu.CompilerParams(
            dimension_semantics=("parallel", "arbitrary")),
    )(f2, f2T, h2aug, adj)

    return out


# speedup vs baseline: 3.0192x; 1.1223x over previous
import functools

import jax
import jax.numpy as jnp
from jax.experimental import pallas as pl
from jax.experimental.pallas import tpu as pltpu

_ALPHA = 0.2


def _layer1_body(nheads, nhid, nclass, bi, x_ref, w_ref, asrc_ref, adst_ref,
                 adj_ref, w2_ref, a2_ref, h2aug_ref, f2_ref, f2T_ref,
                 haug_s, fsrc_s, fdstT_s):
    i = pl.program_id(0)
    w = nhid + 1

    @pl.when(i == 0)
    def _prep():
        h = jnp.dot(x_ref[...], w_ref[...], preferred_element_type=jnp.float32)
        fsrc_s[...] = jnp.dot(
            h, asrc_ref[...],
            preferred_element_type=jnp.float32).astype(jnp.bfloat16)
        fdstT_s[...] = jnp.transpose(
            jnp.dot(h, adst_ref[...],
                    preferred_element_type=jnp.float32)).astype(jnp.bfloat16)
        ones = jnp.ones((h.shape[0], 1), jnp.bfloat16)
        for hd in range(nheads):
            haug_s[:, hd * w:hd * w + nhid] = (
                h[:, hd * nhid:(hd + 1) * nhid].astype(jnp.bfloat16))
            haug_s[:, hd * w + nhid:hd * w + nhid + 1] = ones

    adjb = adj_ref[...].astype(jnp.bfloat16)
    haug = haug_s[...]
    fsrc = fsrc_s[pl.ds(i * bi, bi), :]
    alpha = jnp.bfloat16(_ALPHA)
    parts = []
    for hd in range(nheads):
        s = fsrc[:, hd:hd + 1] + fdstT_s[hd:hd + 1, :]
        m = jnp.minimum(s, alpha * s)
        p = jnp.exp(m) * adjb
        parts.append(jnp.dot(p, haug[:, hd * w:(hd + 1) * w],
                             preferred_element_type=jnp.float32))

    cols = []
    for hd in range(nheads):
        hp = parts[hd][:, 0:nhid]
        rs = parts[hd][:, nhid:nhid + 1]
        x = hp / rs
        cols.append(jnp.where(x > 0.0, x, jnp.exp(x) - 1.0))
    x1 = jnp.concatenate(cols, axis=1)
    h2 = jnp.dot(x1, w2_ref[...], preferred_element_type=jnp.float32)
    f2 = jnp.dot(h2, a2_ref[...], preferred_element_type=jnp.float32)
    f2_ref[...] = f2.astype(jnp.bfloat16)
    f2T_ref[...] = jnp.transpose(f2).astype(jnp.bfloat16)
    h2aug_ref[:, 0:nclass] = h2.astype(jnp.bfloat16)
    h2aug_ref[:, nclass:nclass + 1] = jnp.ones((h2.shape[0], 1), jnp.bfloat16)


def _layer2_body(nclass, f2_ref, f2T_ref, haug_ref, adj_ref, out_ref):
    adjb = adj_ref[...].astype(jnp.bfloat16)
    alpha = jnp.bfloat16(_ALPHA)
    s = f2_ref[:, 0:1] + f2T_ref[1:2, :]
    m = jnp.minimum(s, alpha * s)
    p = jnp.exp(m) * adjb
    acc = jnp.dot(p, haug_ref[...], preferred_element_type=jnp.float32)
    x = acc[:, 0:nclass] / acc[:, nclass:nclass + 1]
    x = jnp.where(x > 0.0, x, jnp.exp(x) - 1.0)
    mx = jnp.max(x, axis=1, keepdims=True)
    lse = mx + jnp.log(jnp.sum(jnp.exp(x - mx), axis=1, keepdims=True))
    out_ref[...] = x - lse


def kernel(features, adj, Ws, As, W_out, a_out):
    n, nfeat = features.shape
    nheads, _, nhid = Ws.shape
    nclass = W_out.shape[1]
    ndim = nheads * nhid
    w1 = nhid + 1

    BI = 1024
    ni = n // BI

    W_all = jnp.transpose(Ws, (1, 0, 2)).reshape(nfeat, ndim)
    eye = jnp.eye(nheads, dtype=jnp.float32)
    a_src_mat = -(eye[:, None, :] * As[:, 0, :nhid][:, :, None]).reshape(ndim, nheads)
    a_dst_mat = -(eye[:, None, :] * As[:, 0, nhid:][:, :, None]).reshape(ndim, nheads)
    a2_mat = -jnp.pad(
        jnp.stack([a_out[0, :nclass], a_out[0, nclass:]], axis=1),
        ((0, 0), (0, 6)))

    h2aug, f2, f2T = pl.pallas_call(
        functools.partial(_layer1_body, nheads, nhid, nclass, BI),
        grid=(ni,),
        in_specs=[
            pl.BlockSpec((n, nfeat), lambda i: (0, 0)),
            pl.BlockSpec((nfeat, ndim), lambda i: (0, 0)),
            pl.BlockSpec((ndim, nheads), lambda i: (0, 0)),
            pl.BlockSpec((ndim, nheads), lambda i: (0, 0)),
            pl.BlockSpec((BI, n), lambda i: (i, 0)),
            pl.BlockSpec((ndim, nclass), lambda i: (0, 0)),
            pl.BlockSpec((nclass, 8), lambda i: (0, 0)),
        ],
        out_specs=[
            pl.BlockSpec((BI, nclass + 1), lambda i: (i, 0)),
            pl.BlockSpec((BI, 8), lambda i: (i, 0)),
            pl.BlockSpec((8, BI), lambda i: (0, i)),
        ],
        out_shape=[
            jax.ShapeDtypeStruct((n, nclass + 1), jnp.bfloat16),
            jax.ShapeDtypeStruct((n, 8), jnp.bfloat16),
            jax.ShapeDtypeStruct((8, n), jnp.bfloat16),
        ],
        scratch_shapes=[
            pltpu.VMEM((n, nheads * w1), jnp.bfloat16),
            pltpu.VMEM((n, nheads), jnp.bfloat16),
            pltpu.VMEM((nheads, n), jnp.bfloat16),
        ],
        compiler_params=pltpu.CompilerParams(
            dimension_semantics=("arbitrary",)),
    )(features, W_all, a_src_mat, a_dst_mat, adj, W_out, a2_mat)

    out = pl.pallas_call(
        functools.partial(_layer2_body, nclass),
        grid=(ni,),
        in_specs=[
            pl.BlockSpec((BI, 8), lambda i: (i, 0)),
            pl.BlockSpec((8, n), lambda i: (0, 0)),
            pl.BlockSpec((n, nclass + 1), lambda i: (0, 0)),
            pl.BlockSpec((BI, n), lambda i: (i, 0)),
        ],
        out_specs=pl.BlockSpec((BI, nclass), lambda i: (i, 0)),
        out_shape=jax.ShapeDtypeStruct((n, nclass), jnp.float32),
        compiler_params=pltpu.CompilerParams(
            dimension_semantics=("arbitrary",)),
    )(f2, f2T, h2aug, adj)

    return out


# speedup vs baseline: 3.1780x; 1.0526x over previous
import functools

import jax
import jax.numpy as jnp
from jax import lax
from jax.experimental import pallas as pl
from jax.experimental.pallas import tpu as pltpu

_ALPHA = 0.2


def _gat_body(nheads, nhid, nclass, bi, ni, x_ref, w_ref, asrc_ref, adst_ref,
              adj_ref, w2_ref, a2_ref, out_ref,
              haug_s, fsrc_s, fdstT_s, h2aug_s, f2_s, f2T_s):
    k = pl.program_id(0)
    w = nhid + 1
    alpha = jnp.bfloat16(_ALPHA)
    adjb = adj_ref[...].astype(jnp.bfloat16)

    @pl.when(k == 0)
    def _prep():
        h = jnp.dot(x_ref[...], w_ref[...], preferred_element_type=jnp.float32)
        fsrc_s[...] = jnp.dot(
            h, asrc_ref[...],
            preferred_element_type=jnp.float32).astype(jnp.bfloat16)
        fdstT_s[...] = jnp.transpose(
            jnp.dot(h, adst_ref[...],
                    preferred_element_type=jnp.float32)).astype(jnp.bfloat16)
        ones = jnp.ones((h.shape[0], 1), jnp.bfloat16)
        for hd in range(nheads):
            haug_s[:, hd * w:hd * w + nhid] = (
                h[:, hd * nhid:(hd + 1) * nhid].astype(jnp.bfloat16))
            haug_s[:, hd * w + nhid:hd * w + nhid + 1] = ones

    @pl.when(k < ni)
    def _layer1():
        haug = haug_s[...]
        fsrc = fsrc_s[pl.ds(k * bi, bi), :]
        parts = []
        for hd in range(nheads):
            s = fsrc[:, hd:hd + 1] + fdstT_s[hd:hd + 1, :]
            m = jnp.minimum(s, alpha * s)
            p = jnp.exp(m) * adjb
            parts.append(jnp.dot(p, haug[:, hd * w:(hd + 1) * w],
                                 preferred_element_type=jnp.float32))
        cols = []
        for hd in range(nheads):
            hp = parts[hd][:, 0:nhid]
            rs = parts[hd][:, nhid:nhid + 1]
            x = hp / rs
            cols.append(jnp.where(x > 0.0, x, jnp.exp(x) - 1.0))
        x1 = jnp.concatenate(cols, axis=1)
        h2 = jnp.dot(x1, w2_ref[...], preferred_element_type=jnp.float32)
        f2 = jnp.dot(h2, a2_ref[...], preferred_element_type=jnp.float32)
        f2_s[pl.ds(k * bi, bi), :] = f2.astype(jnp.bfloat16)
        f2T_s[k] = jnp.transpose(f2).astype(jnp.bfloat16)
        h2aug_s[pl.ds(k * bi, bi), 0:nclass] = h2.astype(jnp.bfloat16)
        h2aug_s[pl.ds(k * bi, bi), nclass:nclass + 1] = jnp.ones(
            (h2.shape[0], 1), jnp.bfloat16)

    @pl.when(k >= ni)
    def _layer2():
        b = k - ni
        f2dT = jnp.concatenate([f2T_s[blk] for blk in range(ni)], axis=1)
        s = f2_s[pl.ds(b * bi, bi), 0:1] + f2dT[1:2, :]
        m = jnp.minimum(s, alpha * s)
        p = jnp.exp(m) * adjb
        acc = jnp.dot(p, h2aug_s[...], preferred_element_type=jnp.float32)
        x = acc[:, 0:nclass] / acc[:, nclass:nclass + 1]
        x = jnp.where(x > 0.0, x, jnp.exp(x) - 1.0)
        mx = jnp.max(x, axis=1, keepdims=True)
        lse = mx + jnp.log(jnp.sum(jnp.exp(x - mx), axis=1, keepdims=True))
        out_ref[...] = x - lse


def kernel(features, adj, Ws, As, W_out, a_out):
    n, nfeat = features.shape
    nheads, _, nhid = Ws.shape
    nclass = W_out.shape[1]
    ndim = nheads * nhid
    w1 = nhid + 1

    BI = 1024
    ni = n // BI

    W_all = jnp.transpose(Ws, (1, 0, 2)).reshape(nfeat, ndim)
    eye = jnp.eye(nheads, dtype=jnp.float32)
    a_src_mat = -(eye[:, None, :] * As[:, 0, :nhid][:, :, None]).reshape(ndim, nheads)
    a_dst_mat = -(eye[:, None, :] * As[:, 0, nhid:][:, :, None]).reshape(ndim, nheads)
    a2_mat = -jnp.pad(
        jnp.stack([a_out[0, :nclass], a_out[0, nclass:]], axis=1),
        ((0, 0), (0, 6)))

    out = pl.pallas_call(
        functools.partial(_gat_body, nheads, nhid, nclass, BI, ni),
        grid=(2 * ni,),
        in_specs=[
            pl.BlockSpec((n, nfeat), lambda k: (0, 0)),
            pl.BlockSpec((nfeat, ndim), lambda k: (0, 0)),
            pl.BlockSpec((ndim, nheads), lambda k: (0, 0)),
            pl.BlockSpec((ndim, nheads), lambda k: (0, 0)),
            pl.BlockSpec((BI, n), lambda k, ni=ni: (lax.rem(k, ni), 0)),
            pl.BlockSpec((ndim, nclass), lambda k: (0, 0)),
            pl.BlockSpec((nclass, 8), lambda k: (0, 0)),
        ],
        out_specs=pl.BlockSpec(
            (BI, nclass),
            lambda k, ni=ni: (jnp.where(k < ni, 0, k - ni), 0)),
        out_shape=jax.ShapeDtypeStruct((n, nclass), jnp.float32),
        scratch_shapes=[
            pltpu.VMEM((n, nheads * w1), jnp.bfloat16),
            pltpu.VMEM((n, nheads), jnp.bfloat16),
            pltpu.VMEM((nheads, n), jnp.bfloat16),
            pltpu.VMEM((n, nclass + 1), jnp.bfloat16),
            pltpu.VMEM((n, 8), jnp.bfloat16),
            pltpu.VMEM((ni, nheads, BI), jnp.bfloat16),
        ],
        compiler_params=pltpu.CompilerParams(
            dimension_semantics=("arbitrary",)),
    )(features, W_all, a_src_mat, a_dst_mat, adj, W_out, a2_mat)

    return out


# speedup vs baseline: 3.3737x; 1.0616x over previous
import functools

import jax
import jax.numpy as jnp
from jax import lax
from jax.experimental import pallas as pl
from jax.experimental.pallas import tpu as pltpu

_ALPHA = 0.2


def _gat_body(nheads, nhid, nclass, bi, ni, x_ref, w_ref, asrc_ref, adst_ref,
              adj_ref, w2_ref, a2_ref, out_ref,
              haug_s, fsrc_s, fdstT_s, h2aug_s, f2_s, f2T_s, adjb_s):
    k = pl.program_id(0)
    w = nhid + 1
    alpha = jnp.bfloat16(_ALPHA)

    @pl.when(k == 0)
    def _prep():
        h = jnp.dot(x_ref[...], w_ref[...], preferred_element_type=jnp.float32)
        fsrc_s[...] = jnp.dot(
            h, asrc_ref[...],
            preferred_element_type=jnp.float32).astype(jnp.bfloat16)
        fdstT_s[...] = jnp.transpose(
            jnp.dot(h, adst_ref[...],
                    preferred_element_type=jnp.float32)).astype(jnp.bfloat16)
        ones = jnp.ones((h.shape[0], 1), jnp.bfloat16)
        for hd in range(nheads):
            haug_s[:, hd * w:hd * w + nhid] = (
                h[:, hd * nhid:(hd + 1) * nhid].astype(jnp.bfloat16))
            haug_s[:, hd * w + nhid:hd * w + nhid + 1] = ones

    @pl.when(k < ni)
    def _layer1():
        adjb = adj_ref[...].astype(jnp.bfloat16)
        adjb_s[pl.ds(k * bi, bi), :] = adjb
        haug = haug_s[...]
        fsrc = fsrc_s[pl.ds(k * bi, bi), :]
        parts = []
        for hd in range(nheads):
            s = fsrc[:, hd:hd + 1] + fdstT_s[hd:hd + 1, :]
            m = jnp.minimum(s, alpha * s)
            p = jnp.exp(m) * adjb
            parts.append(jnp.dot(p, haug[:, hd * w:(hd + 1) * w],
                                 preferred_element_type=jnp.float32))
        cols = []
        for hd in range(nheads):
            hp = parts[hd][:, 0:nhid]
            rs = parts[hd][:, nhid:nhid + 1]
            x = hp / rs
            cols.append(jnp.where(x > 0.0, x, jnp.exp(x) - 1.0))
        x1 = jnp.concatenate(cols, axis=1)
        h2 = jnp.dot(x1, w2_ref[...], preferred_element_type=jnp.float32)
        f2 = jnp.dot(h2, a2_ref[...], preferred_element_type=jnp.float32)
        f2_s[pl.ds(k * bi, bi), :] = f2.astype(jnp.bfloat16)
        f2T_s[k] = jnp.transpose(f2).astype(jnp.bfloat16)
        h2aug_s[pl.ds(k * bi, bi), 0:nclass] = h2.astype(jnp.bfloat16)
        h2aug_s[pl.ds(k * bi, bi), nclass:nclass + 1] = jnp.ones(
            (h2.shape[0], 1), jnp.bfloat16)

    @pl.when(k >= ni)
    def _layer2():
        b = k - ni
        adjb = adjb_s[pl.ds(b * bi, bi), :]
        f2dT = jnp.concatenate([f2T_s[blk] for blk in range(ni)], axis=1)
        s = f2_s[pl.ds(b * bi, bi), 0:1] + f2dT[1:2, :]
        m = jnp.minimum(s, alpha * s)
        p = jnp.exp(m) * adjb
        acc = jnp.dot(p, h2aug_s[...], preferred_element_type=jnp.float32)
        x = acc[:, 0:nclass] / acc[:, nclass:nclass + 1]
        x = jnp.where(x > 0.0, x, jnp.exp(x) - 1.0)
        mx = jnp.max(x, axis=1, keepdims=True)
        lse = mx + jnp.log(jnp.sum(jnp.exp(x - mx), axis=1, keepdims=True))
        out_ref[...] = x - lse


def kernel(features, adj, Ws, As, W_out, a_out):
    n, nfeat = features.shape
    nheads, _, nhid = Ws.shape
    nclass = W_out.shape[1]
    ndim = nheads * nhid
    w1 = nhid + 1

    BI = 1024
    ni = n // BI

    W_all = jnp.transpose(Ws, (1, 0, 2)).reshape(nfeat, ndim)
    eye = jnp.eye(nheads, dtype=jnp.float32)
    a_src_mat = -(eye[:, None, :] * As[:, 0, :nhid][:, :, None]).reshape(ndim, nheads)
    a_dst_mat = -(eye[:, None, :] * As[:, 0, nhid:][:, :, None]).reshape(ndim, nheads)
    a2_mat = -jnp.pad(
        jnp.stack([a_out[0, :nclass], a_out[0, nclass:]], axis=1),
        ((0, 0), (0, 6)))

    out = pl.pallas_call(
        functools.partial(_gat_body, nheads, nhid, nclass, BI, ni),
        grid=(2 * ni,),
        in_specs=[
            pl.BlockSpec((n, nfeat), lambda k: (0, 0)),
            pl.BlockSpec((nfeat, ndim), lambda k: (0, 0)),
            pl.BlockSpec((ndim, nheads), lambda k: (0, 0)),
            pl.BlockSpec((ndim, nheads), lambda k: (0, 0)),
            pl.BlockSpec(
                (BI, n), lambda k, ni=ni: (jnp.where(k < ni, k, ni - 1), 0)),
            pl.BlockSpec((ndim, nclass), lambda k: (0, 0)),
            pl.BlockSpec((nclass, 8), lambda k: (0, 0)),
        ],
        out_specs=pl.BlockSpec(
            (BI, nclass),
            lambda k, ni=ni: (jnp.where(k < ni, 0, k - ni), 0)),
        out_shape=jax.ShapeDtypeStruct((n, nclass), jnp.float32),
        scratch_shapes=[
            pltpu.VMEM((n, nheads * w1), jnp.bfloat16),
            pltpu.VMEM((n, nheads), jnp.bfloat16),
            pltpu.VMEM((nheads, n), jnp.bfloat16),
            pltpu.VMEM((n, nclass + 1), jnp.bfloat16),
            pltpu.VMEM((n, 8), jnp.bfloat16),
            pltpu.VMEM((ni, nheads, BI), jnp.bfloat16),
            pltpu.VMEM((n, n), jnp.bfloat16),
        ],
        compiler_params=pltpu.CompilerParams(
            dimension_semantics=("arbitrary",)),
    )(features, W_all, a_src_mat, a_dst_mat, adj, W_out, a2_mat)

    return out
